# baseline (speedup 1.0000x reference)
import jax
import jax.numpy as jnp
from jax import lax
from jax.experimental import pallas as pl

NUM_CLASSES = 80
N = 134400
BLOCK = 8960


def _criterion_block(pred_cls_ref, rows_ref, labels_col_ref, labels_row_ref,
                     cls_ref, reg_ref, box_ref, npos_ref):
    i = pl.program_id(0)

    @pl.when(i == 0)
    def _init():
        cls_ref[...] = jnp.zeros_like(cls_ref)
        reg_ref[...] = jnp.zeros_like(reg_ref)
        box_ref[...] = jnp.zeros_like(box_ref)
        npos_ref[...] = jnp.zeros_like(npos_ref)

    labels = labels_col_ref[...]
    posb = (labels >= 0) & (labels < NUM_CLASSES)
    x = pred_cls_ref[...]
    col = lax.broadcasted_iota(jnp.int32, x.shape, 1)
    m = (col == labels) & posb
    mf = m.astype(jnp.float32)
    e = jnp.exp(-jnp.abs(x))
    d = 1.0 + e
    r = 1.0 / d
    er = e * r
    ell = jnp.log(d)
    xneg = x < 0.0
    sg = jnp.where(m ^ xneg, er, r)
    sp = jnp.maximum(x, 0.0) - x * mf + ell
    alpha_t = 0.75 - 0.5 * mf
    cls_sum = jnp.sum(alpha_t * sg * sg * sp)

    rows = rows_ref[...]
    prx, pry, prw, prh = (rows[0:1], rows[1:2], rows[2:3], rows[3:4])
    px1, py1, px2, py2 = (rows[4:5], rows[5:6], rows[6:7], rows[7:8])
    gx1, gy1, gx2, gy2 = (rows[8:9], rows[9:10], rows[10:11], rows[11:12])
    ax, ay, aw, ah = (rows[12:13], rows[13:14], rows[14:15], rows[15:16])
    lrow = labels_row_ref[...]
    pos_f = ((lrow >= 0) & (lrow < NUM_CLASSES)).astype(jnp.float32)

    iw = jnp.clip(jnp.minimum(px2, gx2) - jnp.maximum(px1, gx1), 0.0)
    ih = jnp.clip(jnp.minimum(py2, gy2) - jnp.maximum(py1, gy1), 0.0)
    inter = iw * ih
    a1 = jnp.clip(px2 - px1, 0.0) * jnp.clip(py2 - py1, 0.0)
    a2 = jnp.clip(gx2 - gx1, 0.0) * jnp.clip(gy2 - gy1, 0.0)
    union = a1 + a2 - inter
    iou = inter / jnp.clip(union, 1e-7)
    cw = jnp.maximum(px2, gx2) - jnp.minimum(px1, gx1)
    ch = jnp.maximum(py2, gy2) - jnp.minimum(py1, gy1)
    area_c = jnp.clip(cw, 0.0) * jnp.clip(ch, 0.0)
    giou = iou - (area_c - union) / jnp.clip(area_c, 1e-7)
    reg_sum = jnp.sum((1.0 - giou) * pos_f)

    gw = jnp.clip(gx2 - gx1, 1e-7)
    gh = jnp.clip(gy2 - gy1, 1e-7)
    ecx = ((gx1 + gx2) * 0.5 - ax) / aw
    ecy = ((gy1 + gy2) * 0.5 - ay) / ah
    ew = jnp.log(gw / aw)
    eh = jnp.log(gh / ah)
    l1 = (jnp.abs(prx - ecx) + jnp.abs(pry - ecy)
          + jnp.abs(prw - ew) + jnp.abs(prh - eh))
    box_sum = jnp.sum(l1 * pos_f)

    cls_ref[...] += cls_sum
    reg_ref[...] += reg_sum
    box_ref[...] += box_sum
    npos_ref[...] += jnp.sum(pos_f)


@jax.jit
def kernel(pred_cls, pred_reg, pred_box, gt_box, anchors, tgt_labels):
    rows = jnp.concatenate([pred_reg, pred_box, gt_box, anchors], axis=1).T
    grid = (N // BLOCK,)
    scalar_spec = pl.BlockSpec((1, 1), lambda i: (0, 0))
    out = pl.pallas_call(
        _criterion_block,
        grid=grid,
        in_specs=[
            pl.BlockSpec((BLOCK, NUM_CLASSES), lambda i: (i, 0)),
            pl.BlockSpec((16, BLOCK), lambda i: (0, i)),
            pl.BlockSpec((BLOCK, 1), lambda i: (i, 0)),
            pl.BlockSpec((1, BLOCK), lambda i: (0, i)),
        ],
        out_specs=(scalar_spec, scalar_spec, scalar_spec, scalar_spec),
        out_shape=tuple(jax.ShapeDtypeStruct((1, 1), jnp.float32)
                        for _ in range(4)),
    )(pred_cls, rows, tgt_labels.reshape(N, 1), tgt_labels.reshape(1, N))
    cls_sum, reg_sum, box_sum, npos = (o[0, 0] for o in out)
    num_fgs = jnp.maximum(npos, 1.0)
    return jnp.stack([cls_sum, reg_sum, box_sum]) / num_fgs


# speedup vs baseline: 1.2929x; 1.2929x over previous
import jax
import jax.numpy as jnp
from jax import lax
from jax.experimental import pallas as pl

NUM_CLASSES = 80
N = 134400
BLOCK = 8960


def _criterion_block(pred_cls_ref, rows_ref, cls_ref, reg_ref, box_ref,
                     npos_ref):
    i = pl.program_id(0)

    @pl.when(i == 0)
    def _init():
        cls_ref[...] = jnp.zeros_like(cls_ref)
        reg_ref[...] = jnp.zeros_like(reg_ref)
        box_ref[...] = jnp.zeros_like(box_ref)
        npos_ref[...] = jnp.zeros_like(npos_ref)

    rows = rows_ref[...]
    prx, pry, prw, prh = (rows[0:1], rows[1:2], rows[2:3], rows[3:4])
    px1, py1, px2, py2 = (rows[4:5], rows[5:6], rows[6:7], rows[7:8])
    gx1, gy1, gx2, gy2 = (rows[8:9], rows[9:10], rows[10:11], rows[11:12])
    ax, ay, aw, ah = (rows[12:13], rows[13:14], rows[14:15], rows[15:16])
    lrow = rows[16:17]
    pos_row = (lrow >= 0.0) & (lrow < float(NUM_CLASSES))
    pos_f = pos_row.astype(jnp.float32)

    labels_f = lrow.reshape(BLOCK, 1)
    labels = labels_f.astype(jnp.int32)
    posb = (labels >= 0) & (labels < NUM_CLASSES)
    x = pred_cls_ref[...]
    col = lax.broadcasted_iota(jnp.int32, x.shape, 1)
    m = (col == labels) & posb
    mf = m.astype(jnp.float32)
    e = jnp.exp(-jnp.abs(x))
    d = 1.0 + e
    r = 1.0 / d
    er = e * r
    ell = jnp.log(d)
    xneg = x < 0.0
    sg = jnp.where(m ^ xneg, er, r)
    sp = jnp.maximum(x, 0.0) - x * mf + ell
    alpha_t = 0.75 - 0.5 * mf
    cls_sum = jnp.sum(alpha_t * sg * sg * sp)

    iw = jnp.clip(jnp.minimum(px2, gx2) - jnp.maximum(px1, gx1), 0.0)
    ih = jnp.clip(jnp.minimum(py2, gy2) - jnp.maximum(py1, gy1), 0.0)
    inter = iw * ih
    a1 = jnp.clip(px2 - px1, 0.0) * jnp.clip(py2 - py1, 0.0)
    a2 = jnp.clip(gx2 - gx1, 0.0) * jnp.clip(gy2 - gy1, 0.0)
    union = a1 + a2 - inter
    iou = inter / jnp.clip(union, 1e-7)
    cw = jnp.maximum(px2, gx2) - jnp.minimum(px1, gx1)
    ch = jnp.maximum(py2, gy2) - jnp.minimum(py1, gy1)
    area_c = jnp.clip(cw, 0.0) * jnp.clip(ch, 0.0)
    giou = iou - (area_c - union) / jnp.clip(area_c, 1e-7)
    reg_sum = jnp.sum((1.0 - giou) * pos_f)

    gw = jnp.clip(gx2 - gx1, 1e-7)
    gh = jnp.clip(gy2 - gy1, 1e-7)
    ecx = ((gx1 + gx2) * 0.5 - ax) / aw
    ecy = ((gy1 + gy2) * 0.5 - ay) / ah
    ew = jnp.log(gw / aw)
    eh = jnp.log(gh / ah)
    l1 = (jnp.abs(prx - ecx) + jnp.abs(pry - ecy)
          + jnp.abs(prw - ew) + jnp.abs(prh - eh))
    box_sum = jnp.sum(l1 * pos_f)

    cls_ref[...] += cls_sum
    reg_ref[...] += reg_sum
    box_ref[...] += box_sum
    npos_ref[...] += jnp.sum(pos_f)


@jax.jit
def kernel(pred_cls, pred_reg, pred_box, gt_box, anchors, tgt_labels):
    lab_f = tgt_labels.astype(jnp.float32)[:, None]
    rows = jnp.concatenate(
        [pred_reg, pred_box, gt_box, anchors, lab_f], axis=1).T
    grid = (N // BLOCK,)
    scalar_spec = pl.BlockSpec((1, 1), lambda i: (0, 0))
    out = pl.pallas_call(
        _criterion_block,
        grid=grid,
        in_specs=[
            pl.BlockSpec((BLOCK, NUM_CLASSES), lambda i: (i, 0)),
            pl.BlockSpec((17, BLOCK), lambda i: (0, i)),
        ],
        out_specs=(scalar_spec, scalar_spec, scalar_spec, scalar_spec),
        out_shape=tuple(jax.ShapeDtypeStruct((1, 1), jnp.float32)
                        for _ in range(4)),
    )(pred_cls, rows)
    cls_sum, reg_sum, box_sum, npos = (o[0, 0] for o in out)
    num_fgs = jnp.maximum(npos, 1.0)
    return jnp.stack([cls_sum, reg_sum, box_sum]) / num_fgs
